# baseline (speedup 1.0000x reference)
import functools

import numpy as np
import jax
import jax.numpy as jnp
from jax import lax
from jax.experimental import pallas as pl
from jax.experimental.pallas import tpu as pltpu
from jax.experimental.pallas import tpu_sc as plsc

_IMG = 608.0
_GRIDS = (76, 38, 19)
_ANCH = np.array(
    [[10, 13], [16, 30], [33, 23], [30, 61], [62, 45], [59, 119],
     [116, 90], [156, 198], [373, 326]], dtype=np.float32).reshape(3, 3, 2)
_SCALED = [_ANCH[i] / (_IMG / g) for i, g in enumerate(_GRIDS)]
_B = 32
_NLANE = 16
_NCORE = 2


def _sc_body(tt_hbm, asn_hbm, tv, av):
    wid = lax.axis_index("s") * _NCORE + lax.axis_index("c")
    chunk = wid // _NLANE
    j = wid % _NLANE
    c16 = chunk * _NLANE

    pltpu.sync_copy(tt_hbm, tv)

    x1 = tv[pl.ds(0 * _B + c16, 16)] / _IMG
    y1 = tv[pl.ds(1 * _B + c16, 16)] / _IMG
    x2 = tv[pl.ds(2 * _B + c16, 16)] / _IMG
    y2 = tv[pl.ds(3 * _B + c16, 16)] / _IMG

    biou, ba_s, gj_s, gi_s, fx_s, fy_s, rw_s, rh_s = ([] for _ in range(8))
    for i, nG in enumerate(_GRIDS):
        g = jnp.float32(float(nG))
        tx1 = x1 * g
        ty1 = y1 * g
        tx2 = x2 * g
        ty2 = y2 * g
        gx = (tx1 + tx2) / 2.0
        gy = (ty1 + ty2) / 2.0
        gw = tx2 - tx1
        gh = ty2 - ty1
        wh_area = gw * gh

        best_i = None
        best_a = jnp.zeros((16,), jnp.int32)
        for a in range(3):
            w1 = np.float32(_SCALED[i][a, 0])
            h1 = np.float32(_SCALED[i][a, 1])
            ua = np.float32(w1 * h1 + np.float32(1e-16))
            inter = (jnp.minimum(jnp.float32(w1), gw) *
                     jnp.minimum(jnp.float32(h1), gh))
            iou = inter / (jnp.float32(ua) + wh_area - inter)
            if a == 0:
                best_i = iou
            else:
                upd = iou > best_i
                best_a = jnp.where(upd, jnp.int32(a), best_a)
                best_i = jnp.maximum(best_i, iou)
        biou.append(best_i)
        ba_s.append(best_a)

        gi = gx.astype(jnp.int32)
        gj = gy.astype(jnp.int32)
        gi_s.append(gi.astype(jnp.float32))
        gj_s.append(gj.astype(jnp.float32))
        fx_s.append(gx - gi.astype(jnp.float32))
        fy_s.append(gy - gj.astype(jnp.float32))

        w0 = float(_SCALED[i][0, 0]); h0 = float(_SCALED[i][0, 1])
        w1f = float(_SCALED[i][1, 0]); h1f = float(_SCALED[i][1, 1])
        w2f = float(_SCALED[i][2, 0]); h2f = float(_SCALED[i][2, 1])
        aw = jnp.where(best_a == 0, jnp.float32(w0),
                       jnp.where(best_a == 1, jnp.float32(w1f),
                                 jnp.float32(w2f)))
        ah = jnp.where(best_a == 0, jnp.float32(h0),
                       jnp.where(best_a == 1, jnp.float32(h1f),
                                 jnp.float32(h2f)))
        rw_s.append(gw / aw)
        rh_s.append(gh / ah)

    ssel = jnp.zeros((16,), jnp.int32)
    sbest = biou[0]
    for i in (1, 2):
        upd = biou[i] > sbest
        ssel = jnp.where(upd, jnp.int32(i), ssel)
        sbest = jnp.maximum(sbest, biou[i])

    def sel3(vs):
        return jnp.where(ssel == 0, vs[0],
                         jnp.where(ssel == 1, vs[1], vs[2]))

    @pl.when(j == 0)
    def _():
        av[0, :] = sel3(fx_s)
        av[1, :] = sel3(fy_s)
        av[2, :] = sel3(rw_s)
        av[3, :] = sel3(rh_s)
        av[4, :] = ssel.astype(jnp.float32)
        av[5, :] = sel3(ba_s).astype(jnp.float32)
        av[6, :] = sel3(gj_s)
        av[7, :] = sel3(gi_s)
        pltpu.sync_copy(av, asn_hbm.at[pl.ds(chunk * 8, 8)])


_sc_assign = functools.partial(
    pl.kernel,
    out_type=jax.ShapeDtypeStruct((16, 16), jnp.float32),
    mesh=plsc.VectorSubcoreMesh(core_axis_name="c", subcore_axis_name="s"),
    scratch_types=[pltpu.VMEM((4 * _B,), jnp.float32),
                   pltpu.VMEM((8, 16), jnp.float32)],
    compiler_params=pltpu.CompilerParams(needs_layout_passes=False),
)(_sc_body)


def _lse_body(c0, c1, c2, o_lse, m_sc, se_sc):
    first = pl.program_id(0) == 0
    x0 = c0[0]
    x1 = c1[0]

    def rowmax2(x):
        return jnp.max(jnp.max(x, axis=0), axis=1, keepdims=True)

    mx = jnp.maximum(rowmax2(x0), rowmax2(x1))
    m_old = jnp.where(first, jnp.float32(-1e30), m_sc[...])
    se_old = jnp.where(first, jnp.float32(0.0), se_sc[...])
    m_new = jnp.maximum(m_old, mx)
    se = se_old * jnp.exp(m_old - m_new)
    se = se + jnp.sum(jnp.sum(jnp.exp(x0 - m_new[None, :, :]), axis=0),
                      axis=1, keepdims=True)
    se = se + jnp.sum(jnp.sum(jnp.exp(x1 - m_new[None, :, :]), axis=0),
                      axis=1, keepdims=True)
    m_sc[...] = m_new
    se_sc[...] = se

    @pl.when(pl.program_id(0) == 2)
    def _():
        X2 = c2[...]
        x2s = [X2[:, :, 4, :], X2[:, :, 9, :], X2[:, :, 14, :]]
        mvf = jnp.max(jnp.max(jnp.maximum(
            jnp.maximum(x2s[0], x2s[1]), x2s[2]), axis=0),
            axis=0, keepdims=True)
        m2 = mvf.reshape(_B, 1)
        m_f = jnp.maximum(m_new, m2)
        mvf2 = m_f.reshape(1, _B)
        se2 = None
        for xs in x2s:
            s = jnp.sum(jnp.sum(jnp.exp(xs - mvf2[None, :, :]), axis=0),
                        axis=0, keepdims=True)
            se2 = s if se2 is None else se2 + s
        se_f = se * jnp.exp(m_new - m_f) + se2.reshape(_B, 1)
        o_lse[...] = jnp.log(se_f) + m_f


def _row_scalars(r, aref):
    chunk, lane = divmod(r, _NLANE)
    base = chunk * 8
    sf = aref[base + 4, lane]
    a5 = aref[base + 5, lane].astype(jnp.int32) * 5
    gj = aref[base + 6, lane].astype(jnp.int32)
    gi = aref[base + 7, lane].astype(jnp.int32)
    return sf, a5, gj, gi


def _mk_copy(r, aref, o0, o1, o2, gs, sem):
    sf, a5, gj, gi = _row_scalars(r, aref)
    si = sf.astype(jnp.int32)
    b8 = (r // 8) * 8
    cps = [
        pltpu.make_async_copy(
            o0.at[pl.ds(a5, 5), gj, pl.ds(b8, 8), :], gs[0].at[r], sem),
        pltpu.make_async_copy(
            o1.at[pl.ds(a5, 5), gj, pl.ds(b8, 8), :], gs[1].at[r], sem),
        pltpu.make_async_copy(
            o2.at[gj, gi, :, :], gs[2].at[r], sem),
    ]
    return [(si == i, cp) for i, cp in enumerate(cps)]


def _comb_body(aref, lse_ref, o0, o1, o2, o_loss, o_conf, o_off,
               g0, g1, g2, sem):
    gs = (g0, g1, g2)
    descs = [_mk_copy(r, aref, o0, o1, o2, gs, sem) for r in range(_B)]
    for row in descs:
        for cond, cp in row:
            pl.when(cond)(cp.start)
    for row in descs:
        for cond, cp in row:
            pl.when(cond)(cp.wait)

    l76 = lax.broadcasted_iota(jnp.int32, (1, 76), 1)
    l38 = lax.broadcasted_iota(jnp.int32, (1, 38), 1)
    ri15 = lax.broadcasted_iota(jnp.int32, (15, _B), 0)
    ci32 = lax.broadcasted_iota(jnp.int32, (15, _B), 1)
    bi = lax.broadcasted_iota(jnp.int32, (_B, 1), 0)

    p = [jnp.zeros((_B, 1), jnp.float32) for _ in range(5)]
    tx = jnp.zeros((_B, 1), jnp.float32)
    ty = jnp.zeros((_B, 1), jnp.float32)
    rw = jnp.zeros((_B, 1), jnp.float32)
    rh = jnp.zeros((_B, 1), jnp.float32)
    for r in range(_B):
        chunk, lane = divmod(r, _NLANE)
        base = chunk * 8
        sf, a5, gj, gi = _row_scalars(r, aref)
        rs = r % 8
        m0 = jnp.where(l76 == gi, 1.0, 0.0)
        m1 = jnp.where(l38 == gi, 1.0, 0.0)
        oh = jnp.where(bi == r, 1.0, 0.0)
        for c in range(5):
            v0 = jnp.sum(g0[r, c, rs:rs + 1, :] * m0)
            v1 = jnp.sum(g1[r, c, rs:rs + 1, :] * m1)
            m2 = jnp.where((ri15 == a5 + c) & (ci32 == r), 1.0, 0.0)
            v2 = jnp.sum(g2[r] * m2)
            val = jnp.where(sf == 0.0, v0, jnp.where(sf == 1.0, v1, v2))
            p[c] = p[c] + val * oh
        tx = tx + aref[base + 0, lane] * oh
        ty = ty + aref[base + 1, lane] * oh
        rw = rw + aref[base + 2, lane] * oh
        rh = rh + aref[base + 3, lane] * oh

    tw = jnp.log(rw + 1e-16)
    th = jnp.log(rh + 1e-16)

    def sig(x):
        return jnp.clip(jax.nn.sigmoid(x), 0.0001, 1.0 - 0.0001)

    off_per = ((sig(p[0]) - tx) ** 2 + (sig(p[1]) - ty) ** 2 +
               (p[2] - tw) ** 2 + (p[3] - th) ** 2)
    off = jnp.sum(off_per, axis=0, keepdims=True) / jnp.float32(_B)
    lc = jnp.sum(lse_ref[...] - p[4], axis=0, keepdims=True) / jnp.float32(_B)
    o_off[...] = off
    o_conf[...] = lc
    o_loss[...] = off + lc


def kernel(out0, out1, out2, targets):
    ttf = targets.T.reshape(-1)
    asn = _sc_assign(ttf)

    o0t = jnp.transpose(out0, (1, 2, 0, 3))
    o1t = jnp.transpose(out1, (1, 2, 0, 3))
    o2t = jnp.transpose(out2, (2, 3, 1, 0))

    lse = pl.pallas_call(
        _lse_body,
        grid=(3,),
        in_specs=[
            pl.BlockSpec((1, 76, _B, 76), lambda i: (4 + 5 * i, 0, 0, 0)),
            pl.BlockSpec((1, 38, _B, 38), lambda i: (4 + 5 * i, 0, 0, 0)),
            pl.BlockSpec((19, 19, 15, _B), lambda i: (0, 0, 0, 0)),
        ],
        out_specs=pl.BlockSpec((_B, 1), lambda i: (0, 0)),
        out_shape=jax.ShapeDtypeStruct((_B, 1), jnp.float32),
        scratch_shapes=[pltpu.VMEM((_B, 1), jnp.float32),
                        pltpu.VMEM((_B, 1), jnp.float32)],
    )(o0t, o1t, o2t)

    loss, lc, off = pl.pallas_call(
        _comb_body,
        grid=(1,),
        in_specs=[pl.BlockSpec(memory_space=pltpu.SMEM),
                  pl.BlockSpec((_B, 1), lambda i: (0, 0)),
                  pl.BlockSpec(memory_space=pl.ANY),
                  pl.BlockSpec(memory_space=pl.ANY),
                  pl.BlockSpec(memory_space=pl.ANY)],
        out_specs=[pl.BlockSpec((1, 1), lambda i: (0, 0))] * 3,
        out_shape=[jax.ShapeDtypeStruct((1, 1), jnp.float32)] * 3,
        scratch_shapes=[pltpu.VMEM((_B, 5, 8, _GRIDS[0]), jnp.float32),
                        pltpu.VMEM((_B, 5, 8, _GRIDS[1]), jnp.float32),
                        pltpu.VMEM((_B, 15, _B), jnp.float32),
                        pltpu.SemaphoreType.DMA],
    )(asn, lse, o0t, o1t, o2t)

    return (loss.reshape(1), lc.reshape(1), off.reshape(1))


# speedup vs baseline: 1.0322x; 1.0322x over previous
import functools

import numpy as np
import jax
import jax.numpy as jnp
from jax import lax
from jax.experimental import pallas as pl
from jax.experimental.pallas import tpu as pltpu
from jax.experimental.pallas import tpu_sc as plsc

_IMG = 608.0
_GRIDS = (76, 38, 19)
_ANCH = np.array(
    [[10, 13], [16, 30], [33, 23], [30, 61], [62, 45], [59, 119],
     [116, 90], [156, 198], [373, 326]], dtype=np.float32).reshape(3, 3, 2)
_SCALED = [_ANCH[i] / (_IMG / g) for i, g in enumerate(_GRIDS)]
_B = 32
_NLANE = 16
_NCORE = 2


def _sc_body(tt_hbm, asn_hbm, tv, av):
    wid = lax.axis_index("s") * _NCORE + lax.axis_index("c")
    chunk = wid // _NLANE
    j = wid % _NLANE
    c16 = chunk * _NLANE

    pltpu.sync_copy(tt_hbm, tv)

    x1 = tv[pl.ds(0 * _B + c16, 16)] / _IMG
    y1 = tv[pl.ds(1 * _B + c16, 16)] / _IMG
    x2 = tv[pl.ds(2 * _B + c16, 16)] / _IMG
    y2 = tv[pl.ds(3 * _B + c16, 16)] / _IMG

    biou, ba_s, gj_s, gi_s, fx_s, fy_s, rw_s, rh_s = ([] for _ in range(8))
    for i, nG in enumerate(_GRIDS):
        g = jnp.float32(float(nG))
        tx1 = x1 * g
        ty1 = y1 * g
        tx2 = x2 * g
        ty2 = y2 * g
        gx = (tx1 + tx2) / 2.0
        gy = (ty1 + ty2) / 2.0
        gw = tx2 - tx1
        gh = ty2 - ty1
        wh_area = gw * gh

        best_i = None
        best_a = jnp.zeros((16,), jnp.int32)
        for a in range(3):
            w1 = np.float32(_SCALED[i][a, 0])
            h1 = np.float32(_SCALED[i][a, 1])
            ua = np.float32(w1 * h1 + np.float32(1e-16))
            inter = (jnp.minimum(jnp.float32(w1), gw) *
                     jnp.minimum(jnp.float32(h1), gh))
            iou = inter / (jnp.float32(ua) + wh_area - inter)
            if a == 0:
                best_i = iou
            else:
                upd = iou > best_i
                best_a = jnp.where(upd, jnp.int32(a), best_a)
                best_i = jnp.maximum(best_i, iou)
        biou.append(best_i)
        ba_s.append(best_a)

        gi = gx.astype(jnp.int32)
        gj = gy.astype(jnp.int32)
        gi_s.append(gi.astype(jnp.float32))
        gj_s.append(gj.astype(jnp.float32))
        fx_s.append(gx - gi.astype(jnp.float32))
        fy_s.append(gy - gj.astype(jnp.float32))

        w0 = float(_SCALED[i][0, 0]); h0 = float(_SCALED[i][0, 1])
        w1f = float(_SCALED[i][1, 0]); h1f = float(_SCALED[i][1, 1])
        w2f = float(_SCALED[i][2, 0]); h2f = float(_SCALED[i][2, 1])
        aw = jnp.where(best_a == 0, jnp.float32(w0),
                       jnp.where(best_a == 1, jnp.float32(w1f),
                                 jnp.float32(w2f)))
        ah = jnp.where(best_a == 0, jnp.float32(h0),
                       jnp.where(best_a == 1, jnp.float32(h1f),
                                 jnp.float32(h2f)))
        rw_s.append(gw / aw)
        rh_s.append(gh / ah)

    ssel = jnp.zeros((16,), jnp.int32)
    sbest = biou[0]
    for i in (1, 2):
        upd = biou[i] > sbest
        ssel = jnp.where(upd, jnp.int32(i), ssel)
        sbest = jnp.maximum(sbest, biou[i])

    def sel3(vs):
        return jnp.where(ssel == 0, vs[0],
                         jnp.where(ssel == 1, vs[1], vs[2]))

    @pl.when(j == 0)
    def _():
        av[0, :] = sel3(fx_s)
        av[1, :] = sel3(fy_s)
        av[2, :] = sel3(rw_s)
        av[3, :] = sel3(rh_s)
        av[4, :] = ssel.astype(jnp.float32)
        av[5, :] = sel3(ba_s).astype(jnp.float32)
        av[6, :] = sel3(gj_s)
        av[7, :] = sel3(gi_s)
        pltpu.sync_copy(av, asn_hbm.at[pl.ds(chunk * 8, 8)])


_sc_assign = functools.partial(
    pl.kernel,
    out_type=jax.ShapeDtypeStruct((16, 16), jnp.float32),
    mesh=plsc.VectorSubcoreMesh(core_axis_name="c", subcore_axis_name="s"),
    scratch_types=[pltpu.VMEM((4 * _B,), jnp.float32),
                   pltpu.VMEM((8, 16), jnp.float32)],
    compiler_params=pltpu.CompilerParams(needs_layout_passes=False),
)(_sc_body)


def _lse_body(c0, c1, c2, o_lse, m_sc, se_sc, x2buf, sem2):
    first = pl.program_id(0) == 0
    x0 = c0[0]
    x1 = c1[0]
    x2cp = pltpu.make_async_copy(c2, x2buf, sem2)
    pl.when(first)(x2cp.start)

    def rowmax2(x):
        return jnp.max(jnp.max(x, axis=0), axis=1, keepdims=True)

    mx = jnp.maximum(rowmax2(x0), rowmax2(x1))
    m_old = jnp.where(first, jnp.float32(-1e30), m_sc[...])
    se_old = jnp.where(first, jnp.float32(0.0), se_sc[...])
    m_new = jnp.maximum(m_old, mx)
    se = se_old * jnp.exp(m_old - m_new)
    se = se + jnp.sum(jnp.sum(jnp.exp(x0 - m_new[None, :, :]), axis=0),
                      axis=1, keepdims=True)
    se = se + jnp.sum(jnp.sum(jnp.exp(x1 - m_new[None, :, :]), axis=0),
                      axis=1, keepdims=True)
    m_sc[...] = m_new
    se_sc[...] = se

    @pl.when(pl.program_id(0) == 2)
    def _():
        x2cp.wait()
        X2 = x2buf[...]
        x2s = [X2[:, :, 4, :], X2[:, :, 9, :], X2[:, :, 14, :]]
        mvf = jnp.max(jnp.max(jnp.maximum(
            jnp.maximum(x2s[0], x2s[1]), x2s[2]), axis=0),
            axis=0, keepdims=True)
        m2 = mvf.reshape(_B, 1)
        m_f = jnp.maximum(m_new, m2)
        mvf2 = m_f.reshape(1, _B)
        se2 = None
        for xs in x2s:
            s = jnp.sum(jnp.sum(jnp.exp(xs - mvf2[None, :, :]), axis=0),
                        axis=0, keepdims=True)
            se2 = s if se2 is None else se2 + s
        se_f = se * jnp.exp(m_new - m_f) + se2.reshape(_B, 1)
        o_lse[...] = jnp.log(se_f) + m_f


def _row_scalars(r, aref):
    chunk, lane = divmod(r, _NLANE)
    base = chunk * 8
    sf = aref[base + 4, lane]
    a5 = aref[base + 5, lane].astype(jnp.int32) * 5
    gj = aref[base + 6, lane].astype(jnp.int32)
    gi = aref[base + 7, lane].astype(jnp.int32)
    return sf, a5, gj, gi


def _mk_copy(r, aref, o0, o1, o2, gs, sem):
    sf, a5, gj, gi = _row_scalars(r, aref)
    si = sf.astype(jnp.int32)
    b8 = (r // 8) * 8
    cps = [
        pltpu.make_async_copy(
            o0.at[pl.ds(a5, 5), gj, pl.ds(b8, 8), :], gs[0].at[r], sem),
        pltpu.make_async_copy(
            o1.at[pl.ds(a5, 5), gj, pl.ds(b8, 8), :], gs[1].at[r], sem),
        pltpu.make_async_copy(
            o2.at[gj, gi, :, :], gs[2].at[r], sem),
    ]
    return [(si == i, cp) for i, cp in enumerate(cps)]


def _comb_body(aref, lse_ref, o0, o1, o2, o_loss, o_conf, o_off,
               g0, g1, g2, sem):
    gs = (g0, g1, g2)
    descs = [_mk_copy(r, aref, o0, o1, o2, gs, sem) for r in range(_B)]
    for row in descs:
        for cond, cp in row:
            pl.when(cond)(cp.start)
    for row in descs:
        for cond, cp in row:
            pl.when(cond)(cp.wait)

    l76 = lax.broadcasted_iota(jnp.int32, (1, 76), 1)
    l38 = lax.broadcasted_iota(jnp.int32, (1, 38), 1)
    ri15 = lax.broadcasted_iota(jnp.int32, (15, _B), 0)
    ci32 = lax.broadcasted_iota(jnp.int32, (15, _B), 1)
    bi = lax.broadcasted_iota(jnp.int32, (_B, 1), 0)

    p = [jnp.zeros((_B, 1), jnp.float32) for _ in range(5)]
    tx = jnp.zeros((_B, 1), jnp.float32)
    ty = jnp.zeros((_B, 1), jnp.float32)
    rw = jnp.zeros((_B, 1), jnp.float32)
    rh = jnp.zeros((_B, 1), jnp.float32)
    for r in range(_B):
        chunk, lane = divmod(r, _NLANE)
        base = chunk * 8
        sf, a5, gj, gi = _row_scalars(r, aref)
        rs = r % 8
        m0 = jnp.where(l76 == gi, 1.0, 0.0)
        m1 = jnp.where(l38 == gi, 1.0, 0.0)
        oh = jnp.where(bi == r, 1.0, 0.0)
        for c in range(5):
            v0 = jnp.sum(g0[r, c, rs:rs + 1, :] * m0)
            v1 = jnp.sum(g1[r, c, rs:rs + 1, :] * m1)
            m2 = jnp.where((ri15 == a5 + c) & (ci32 == r), 1.0, 0.0)
            v2 = jnp.sum(g2[r] * m2)
            val = jnp.where(sf == 0.0, v0, jnp.where(sf == 1.0, v1, v2))
            p[c] = p[c] + val * oh
        tx = tx + aref[base + 0, lane] * oh
        ty = ty + aref[base + 1, lane] * oh
        rw = rw + aref[base + 2, lane] * oh
        rh = rh + aref[base + 3, lane] * oh

    tw = jnp.log(rw + 1e-16)
    th = jnp.log(rh + 1e-16)

    def sig(x):
        return jnp.clip(jax.nn.sigmoid(x), 0.0001, 1.0 - 0.0001)

    off_per = ((sig(p[0]) - tx) ** 2 + (sig(p[1]) - ty) ** 2 +
               (p[2] - tw) ** 2 + (p[3] - th) ** 2)
    off = jnp.sum(off_per, axis=0, keepdims=True) / jnp.float32(_B)
    lc = jnp.sum(lse_ref[...] - p[4], axis=0, keepdims=True) / jnp.float32(_B)
    o_off[...] = off
    o_conf[...] = lc
    o_loss[...] = off + lc


def kernel(out0, out1, out2, targets):
    ttf = targets.T.reshape(-1)
    asn = _sc_assign(ttf)

    o0t = jnp.transpose(out0, (1, 2, 0, 3))
    o1t = jnp.transpose(out1, (1, 2, 0, 3))
    o2t = jnp.transpose(out2, (2, 3, 1, 0))

    lse = pl.pallas_call(
        _lse_body,
        grid=(3,),
        in_specs=[
            pl.BlockSpec((1, 76, _B, 76), lambda i: (4 + 5 * i, 0, 0, 0)),
            pl.BlockSpec((1, 38, _B, 38), lambda i: (4 + 5 * i, 0, 0, 0)),
            pl.BlockSpec(memory_space=pl.ANY),
        ],
        out_specs=pl.BlockSpec((_B, 1), lambda i: (0, 0)),
        out_shape=jax.ShapeDtypeStruct((_B, 1), jnp.float32),
        scratch_shapes=[pltpu.VMEM((_B, 1), jnp.float32),
                        pltpu.VMEM((_B, 1), jnp.float32),
                        pltpu.VMEM((19, 19, 15, _B), jnp.float32),
                        pltpu.SemaphoreType.DMA],
    )(o0t, o1t, o2t)

    loss, lc, off = pl.pallas_call(
        _comb_body,
        grid=(1,),
        in_specs=[pl.BlockSpec(memory_space=pltpu.SMEM),
                  pl.BlockSpec((_B, 1), lambda i: (0, 0)),
                  pl.BlockSpec(memory_space=pl.ANY),
                  pl.BlockSpec(memory_space=pl.ANY),
                  pl.BlockSpec(memory_space=pl.ANY)],
        out_specs=[pl.BlockSpec((1, 1), lambda i: (0, 0))] * 3,
        out_shape=[jax.ShapeDtypeStruct((1, 1), jnp.float32)] * 3,
        scratch_shapes=[pltpu.VMEM((_B, 5, 8, _GRIDS[0]), jnp.float32),
                        pltpu.VMEM((_B, 5, 8, _GRIDS[1]), jnp.float32),
                        pltpu.VMEM((_B, 15, _B), jnp.float32),
                        pltpu.SemaphoreType.DMA],
    )(asn, lse, o0t, o1t, o2t)

    return (loss.reshape(1), lc.reshape(1), off.reshape(1))
